# baseline (speedup 1.0000x reference)
import functools

import jax
import jax.numpy as jnp
from jax import lax
from jax.experimental import pallas as pl
from jax.experimental.pallas import tpu as pltpu
from jax.experimental.pallas import tpu_sc as plsc

_DIM = 64
_B = 4096
_L = 200
_LP = 208
_HALF = 104
_NC, _NS, _LANES = 2, 16, 16
_NW = _NC * _NS
_BPW = _B // _NW
_H1, _H2, _OUTP = 256, 128, 128
_NG = _DIM // _LANES


@functools.partial(
    pl.kernel,
    mesh=plsc.VectorSubcoreMesh(core_axis_name="c", subcore_axis_name="s"),
    compiler_params=pltpu.CompilerParams(use_tc_tiling_on_sc=False),
    out_type=jax.ShapeDtypeStruct((_B, _DIM), jnp.float32),
    scratch_types=[
        pltpu.VMEM((_BPW, _LP), jnp.int32),
        pltpu.VMEM((_LP, _DIM), jnp.float32),
        pltpu.VMEM((_BPW, _DIM), jnp.float32),
        pltpu.SemaphoreType.DMA,
    ],
)
def _sc_sum(x_hbm, table_hbm, out_hbm, idx_v, rows_v, sums_v, sem):
    wid = lax.axis_index("s") * _NC + lax.axis_index("c")
    base = wid * _BPW
    pltpu.sync_copy(x_hbm.at[pl.ds(base, _BPW)], idx_v)

    def bag(b, carry):
        c0 = pltpu.async_copy(table_hbm.at[idx_v.at[b, pl.ds(0, _HALF)]],
                              rows_v.at[pl.ds(0, _HALF)], sem)
        c1 = pltpu.async_copy(table_hbm.at[idx_v.at[b, pl.ds(_HALF, _HALF)]],
                              rows_v.at[pl.ds(_HALF, _HALF)], sem)
        c0.wait()
        c1.wait()

        def srow(j, accs):
            out = []
            for g in range(_NG):
                a = accs[g]
                for jj in range(4):
                    a = a + rows_v[j * 4 + jj, pl.ds(g * _LANES, _LANES)]
                out.append(a)
            return tuple(out)

        accs = lax.fori_loop(
            0, _LP // 4, srow,
            tuple(jnp.zeros((_LANES,), jnp.float32) for _ in range(_NG)))
        for g in range(_NG):
            sums_v[b, pl.ds(g * _LANES, _LANES)] = accs[g]
        return carry

    lax.fori_loop(0, _BPW, bag, 0)
    pltpu.sync_copy(sums_v, out_hbm.at[pl.ds(base, _BPW)])


def _mlp_body(s_ref, x_ref, w1_ref, b1_ref, w2_ref, b2_ref, w3_ref, b3_ref,
              o_ref):
    hp = lax.Precision.HIGHEST
    mask = (x_ref[...] != 0).astype(jnp.float32)
    lengths = jnp.maximum(jnp.sum(mask, axis=1, keepdims=True), 1.0)
    pooled = s_ref[...] / lengths
    h = jnp.dot(pooled, w1_ref[...], precision=hp,
                preferred_element_type=jnp.float32) + b1_ref[...]
    h = jnp.maximum(h, 0.0)
    h = jnp.dot(h, w2_ref[...], precision=hp,
                preferred_element_type=jnp.float32) + b2_ref[...]
    h = jnp.maximum(h, 0.0)
    o_ref[...] = jnp.dot(h, w3_ref[...], precision=hp,
                         preferred_element_type=jnp.float32) + b3_ref[...]


def kernel(x, table, W1, b1, W2, b2, W3, b3):
    x_pad = jnp.pad(x, ((0, 0), (0, _LP - _L)))
    sums = _sc_sum(x_pad, table)

    nblk = 4
    bm = _B // nblk
    out = pl.pallas_call(
        _mlp_body,
        grid=(nblk,),
        in_specs=[
            pl.BlockSpec((bm, _DIM), lambda i: (i, 0)),
            pl.BlockSpec((bm, _L), lambda i: (i, 0)),
            pl.BlockSpec((_DIM, _H1), lambda i: (0, 0)),
            pl.BlockSpec((1, _H1), lambda i: (0, 0)),
            pl.BlockSpec((_H1, _H2), lambda i: (0, 0)),
            pl.BlockSpec((1, _H2), lambda i: (0, 0)),
            pl.BlockSpec((_H2, _OUTP), lambda i: (0, 0)),
            pl.BlockSpec((1, _OUTP), lambda i: (0, 0)),
        ],
        out_specs=pl.BlockSpec((bm, _OUTP), lambda i: (i, 0)),
        out_shape=jax.ShapeDtypeStruct((_B, _OUTP), jnp.float32),
    )(sums, x, W1, b1.reshape(1, _H1), W2, b2.reshape(1, _H2),
      jnp.pad(W3, ((0, 0), (0, _OUTP - W3.shape[1]))),
      jnp.pad(b3, (0, _OUTP - b3.shape[0])).reshape(1, _OUTP))
    return out[:, :2]


# speedup vs baseline: 1.8658x; 1.8658x over previous
import functools

import jax
import jax.numpy as jnp
from jax import lax
from jax.experimental import pallas as pl
from jax.experimental.pallas import tpu as pltpu
from jax.experimental.pallas import tpu_sc as plsc

_DIM = 64
_B = 4096
_L = 200
_NC, _NS, _LANES = 2, 16, 16
_NW = _NC * _NS
_BPW = _B // _NW
_H1, _H2, _OUTP = 256, 128, 128
_NG = _DIM // _LANES
_CH = 4
_ROWS = _CH * _L
_NCHUNK = _BPW // _CH


@functools.partial(
    pl.kernel,
    mesh=plsc.VectorSubcoreMesh(core_axis_name="c", subcore_axis_name="s"),
    compiler_params=pltpu.CompilerParams(use_tc_tiling_on_sc=False),
    out_type=jax.ShapeDtypeStruct((_B, _DIM), jnp.float32),
    scratch_types=[
        pltpu.VMEM((2, _ROWS), jnp.int32),
        pltpu.VMEM((2, _ROWS, _DIM), jnp.float32),
        pltpu.VMEM((_BPW, _DIM), jnp.float32),
        pltpu.SemaphoreType.DMA,
        pltpu.SemaphoreType.DMA,
    ],
)
def _sc_sum(x_hbm, table_hbm, out_hbm, idx_v, rows_v, sums_v, sem0, sem1):
    wid = lax.axis_index("s") * _NC + lax.axis_index("c")
    base = wid * _BPW
    xbase = base * _L
    sems = (sem0, sem1)

    def start(k, buf):
        pltpu.sync_copy(x_hbm.at[pl.ds(xbase + k * _ROWS, _ROWS)],
                        idx_v.at[buf])
        pltpu.async_copy(table_hbm.at[idx_v.at[buf]], rows_v.at[buf],
                         sems[buf])

    def wait(buf):
        pltpu.make_async_copy(table_hbm.at[idx_v.at[buf]], rows_v.at[buf],
                              sems[buf]).wait()

    def sum_chunk(k, buf):
        def srow(j, accs):
            out = []
            for c in range(_CH):
                for g in range(_NG):
                    a = accs[c * _NG + g]
                    a = a + rows_v[buf, c * _L + j, pl.ds(g * _LANES, _LANES)]
                    out.append(a)
            return tuple(out)

        accs = lax.fori_loop(
            0, _L, srow,
            tuple(jnp.zeros((_LANES,), jnp.float32)
                  for _ in range(_CH * _NG)))
        for c in range(_CH):
            for g in range(_NG):
                sums_v[k * _CH + c, pl.ds(g * _LANES, _LANES)] = \
                    accs[c * _NG + g]

    start(0, 0)

    def body(t, carry):
        start(2 * t + 1, 1)
        wait(0)
        sum_chunk(2 * t, 0)

        @pl.when(t < _NCHUNK // 2 - 1)
        def _():
            start(2 * t + 2, 0)

        wait(1)
        sum_chunk(2 * t + 1, 1)
        return carry

    lax.fori_loop(0, _NCHUNK // 2, body, 0)
    pltpu.sync_copy(sums_v, out_hbm.at[pl.ds(base, _BPW)])


def _mlp_body(s_ref, x_ref, w1_ref, b1_ref, w2_ref, b2_ref, w3_ref, b3_ref,
              o_ref):
    hp = lax.Precision.HIGHEST
    mask = (x_ref[...] != 0).astype(jnp.float32)
    lengths = jnp.maximum(jnp.sum(mask, axis=1, keepdims=True), 1.0)
    pooled = s_ref[...] / lengths
    h = jnp.dot(pooled, w1_ref[...], precision=hp,
                preferred_element_type=jnp.float32) + b1_ref[...]
    h = jnp.maximum(h, 0.0)
    h = jnp.dot(h, w2_ref[...], precision=hp,
                preferred_element_type=jnp.float32) + b2_ref[...]
    h = jnp.maximum(h, 0.0)
    o_ref[...] = jnp.dot(h, w3_ref[...], precision=hp,
                         preferred_element_type=jnp.float32) + b3_ref[...]


def kernel(x, table, W1, b1, W2, b2, W3, b3):
    sums = _sc_sum(x.reshape(-1), table)

    nblk = 4
    bm = _B // nblk
    out = pl.pallas_call(
        _mlp_body,
        grid=(nblk,),
        in_specs=[
            pl.BlockSpec((bm, _DIM), lambda i: (i, 0)),
            pl.BlockSpec((bm, _L), lambda i: (i, 0)),
            pl.BlockSpec((_DIM, _H1), lambda i: (0, 0)),
            pl.BlockSpec((1, _H1), lambda i: (0, 0)),
            pl.BlockSpec((_H1, _H2), lambda i: (0, 0)),
            pl.BlockSpec((1, _H2), lambda i: (0, 0)),
            pl.BlockSpec((_H2, _OUTP), lambda i: (0, 0)),
            pl.BlockSpec((1, _OUTP), lambda i: (0, 0)),
        ],
        out_specs=pl.BlockSpec((bm, _OUTP), lambda i: (i, 0)),
        out_shape=jax.ShapeDtypeStruct((_B, _OUTP), jnp.float32),
    )(sums, x, W1, b1.reshape(1, _H1), W2, b2.reshape(1, _H2),
      jnp.pad(W3, ((0, 0), (0, _OUTP - W3.shape[1]))),
      jnp.pad(b3, (0, _OUTP - b3.shape[0])).reshape(1, _OUTP))
    return out[:, :2]


# speedup vs baseline: 2.0762x; 1.1128x over previous
import functools

import jax
import jax.numpy as jnp
from jax import lax
from jax.experimental import pallas as pl
from jax.experimental.pallas import tpu as pltpu
from jax.experimental.pallas import tpu_sc as plsc

_DIM = 64
_B = 4096
_L = 200
_V = 1000000
_NC, _NS, _LANES = 2, 16, 16
_NW = _NC * _NS
_BPW = _B // _NW
_H1, _H2, _OUTP = 256, 128, 128
_NG = _DIM // _LANES
_CH = 4
_ROWS = _CH * _L
_NCHUNK = _BPW // _CH

_HALFV = 512000
_BQ = 4096
_NBLK = _HALFV // _BQ
_LASTB = (_V + _BQ - 1) // _BQ - 1


def _conv_body(a_ref, b_ref, o_ref):
    eye = (lax.broadcasted_iota(jnp.int32, (_DIM, _DIM), 0) ==
           lax.broadcasted_iota(jnp.int32, (_DIM, _DIM), 1)
           ).astype(jnp.float32)
    dn = (((0,), (0,)), ((), ()))
    hp = lax.Precision.HIGHEST
    t1 = lax.dot_general(a_ref[...], eye, dn, precision=hp,
                         preferred_element_type=jnp.float32)
    t2 = lax.dot_general(b_ref[...], eye, dn, precision=hp,
                         preferred_element_type=jnp.float32)
    o_ref[...] = jnp.concatenate([t1, t2], axis=1)


@functools.partial(
    pl.kernel,
    mesh=plsc.VectorSubcoreMesh(core_axis_name="c", subcore_axis_name="s"),
    compiler_params=pltpu.CompilerParams(use_tc_tiling_on_sc=False),
    out_type=jax.ShapeDtypeStruct((_B, _DIM), jnp.float32),
    scratch_types=[
        pltpu.VMEM((2, _ROWS), jnp.int32),
        pltpu.VMEM((2, _ROWS, _DIM), jnp.float32),
        pltpu.VMEM((_BPW, _DIM), jnp.float32),
        pltpu.SemaphoreType.DMA,
        pltpu.SemaphoreType.DMA,
    ],
)
def _sc_sum(x_hbm, table_hbm, out_hbm, idx_v, rows_v, sums_v, sem0, sem1):
    wid = lax.axis_index("s") * _NC + lax.axis_index("c")
    base = wid * _BPW
    xbase = base * _L
    sems = (sem0, sem1)

    def start(k, buf):
        pltpu.sync_copy(x_hbm.at[pl.ds(xbase + k * _ROWS, _ROWS)],
                        idx_v.at[buf])

        def remap(i, carry):
            v = idx_v[buf, pl.ds(i * _LANES, _LANES)]
            idx_v[buf, pl.ds(i * _LANES, _LANES)] = jnp.where(
                v < _HALFV, 2 * v, 2 * v - (2 * _HALFV - 1))
            return carry

        lax.fori_loop(0, _ROWS // _LANES, remap, 0)
        pltpu.async_copy(table_hbm.at[idx_v.at[buf]], rows_v.at[buf],
                         sems[buf])

    def wait(buf):
        pltpu.make_async_copy(table_hbm.at[idx_v.at[buf]], rows_v.at[buf],
                              sems[buf]).wait()

    def sum_chunk(k, buf):
        def srow(j, accs):
            out = []
            for c in range(_CH):
                for g in range(_NG):
                    a = accs[c * _NG + g]
                    a = a + rows_v[buf, c * _L + j, pl.ds(g * _LANES, _LANES)]
                    out.append(a)
            return tuple(out)

        accs = lax.fori_loop(
            0, _L, srow,
            tuple(jnp.zeros((_LANES,), jnp.float32)
                  for _ in range(_CH * _NG)))
        for c in range(_CH):
            for g in range(_NG):
                sums_v[k * _CH + c, pl.ds(g * _LANES, _LANES)] = \
                    accs[c * _NG + g]

    start(0, 0)

    def body(t, carry):
        start(2 * t + 1, 1)
        wait(0)
        sum_chunk(2 * t, 0)

        @pl.when(t < _NCHUNK // 2 - 1)
        def _():
            start(2 * t + 2, 0)

        wait(1)
        sum_chunk(2 * t + 1, 1)
        return carry

    lax.fori_loop(0, _NCHUNK // 2, body, 0)
    pltpu.sync_copy(sums_v, out_hbm.at[pl.ds(base, _BPW)])


def _mlp_body(s_ref, x_ref, w1_ref, b1_ref, w2_ref, b2_ref, w3_ref, b3_ref,
              o_ref):
    hp = lax.Precision.HIGHEST
    mask = (x_ref[...] != 0).astype(jnp.float32)
    lengths = jnp.maximum(jnp.sum(mask, axis=1, keepdims=True), 1.0)
    pooled = s_ref[...] / lengths
    h = jnp.dot(pooled, w1_ref[...], precision=hp,
                preferred_element_type=jnp.float32) + b1_ref[...]
    h = jnp.maximum(h, 0.0)
    h = jnp.dot(h, w2_ref[...], precision=hp,
                preferred_element_type=jnp.float32) + b2_ref[...]
    h = jnp.maximum(h, 0.0)
    o_ref[...] = jnp.dot(h, w3_ref[...], precision=hp,
                         preferred_element_type=jnp.float32) + b3_ref[...]


def kernel(x, table, W1, b1, W2, b2, W3, b3):
    tableT = table.T
    conv = pl.pallas_call(
        _conv_body,
        grid=(_NBLK,),
        in_specs=[
            pl.BlockSpec((_DIM, _BQ), lambda i: (0, i)),
            pl.BlockSpec((_DIM, _BQ),
                         lambda i: (0, jnp.minimum(i + _NBLK, _LASTB))),
        ],
        out_specs=pl.BlockSpec((_BQ, 2 * _DIM), lambda i: (i, 0)),
        out_shape=jax.ShapeDtypeStruct((_HALFV, 2 * _DIM), jnp.float32),
    )(tableT, tableT)
    packed = conv.reshape(2 * _HALFV, _DIM)

    sums = _sc_sum(x.reshape(-1), packed)

    nblk = 4
    bm = _B // nblk
    out = pl.pallas_call(
        _mlp_body,
        grid=(nblk,),
        in_specs=[
            pl.BlockSpec((bm, _DIM), lambda i: (i, 0)),
            pl.BlockSpec((bm, _L), lambda i: (i, 0)),
            pl.BlockSpec((_DIM, _H1), lambda i: (0, 0)),
            pl.BlockSpec((1, _H1), lambda i: (0, 0)),
            pl.BlockSpec((_H1, _H2), lambda i: (0, 0)),
            pl.BlockSpec((1, _H2), lambda i: (0, 0)),
            pl.BlockSpec((_H2, _OUTP), lambda i: (0, 0)),
            pl.BlockSpec((1, _OUTP), lambda i: (0, 0)),
        ],
        out_specs=pl.BlockSpec((bm, _OUTP), lambda i: (i, 0)),
        out_shape=jax.ShapeDtypeStruct((_B, _OUTP), jnp.float32),
    )(sums, x, W1, b1.reshape(1, _H1), W2, b2.reshape(1, _H2),
      jnp.pad(W3, ((0, 0), (0, _OUTP - W3.shape[1]))),
      jnp.pad(b3, (0, _OUTP - b3.shape[0])).reshape(1, _OUTP))
    return out[:, :2]


# speedup vs baseline: 3.9993x; 1.9263x over previous
import functools

import jax
import jax.numpy as jnp
from jax import lax
from jax.experimental import pallas as pl
from jax.experimental.pallas import tpu as pltpu
from jax.experimental.pallas import tpu_sc as plsc

_DIM = 64
_B = 4096
_L = 200
_V = 1000000
_NC, _NS, _LANES = 2, 16, 16
_NW = _NC * _NS
_BPW = _B // _NW
_H1, _H2, _OUTP = 256, 128, 128
_NG = _DIM // _LANES
_CH = 4
_ROWS = _CH * _L
_NCHUNK = _BPW // _CH

_HALFV = 512000
_BQ = 4096
_NBLK = _HALFV // _BQ
_LASTB = (_V + _BQ - 1) // _BQ - 1


def _conv_body(a_ref, b_ref, o_ref):
    two = 2 * _DIM
    eye = (lax.broadcasted_iota(jnp.int32, (two, two), 0) ==
           lax.broadcasted_iota(jnp.int32, (two, two), 1)
           ).astype(jnp.float32)
    dn = (((0,), (0,)), ((), ()))
    hp = lax.Precision.DEFAULT
    s = jnp.concatenate([a_ref[...], b_ref[...]], axis=0)
    o_ref[...] = lax.dot_general(s, eye, dn, precision=hp,
                                 preferred_element_type=jnp.float32)


@functools.partial(
    pl.kernel,
    mesh=plsc.VectorSubcoreMesh(core_axis_name="c", subcore_axis_name="s"),
    compiler_params=pltpu.CompilerParams(use_tc_tiling_on_sc=False),
    out_type=jax.ShapeDtypeStruct((_B, _DIM), jnp.float32),
    scratch_types=[
        pltpu.VMEM((2, _ROWS), jnp.int32),
        pltpu.VMEM((2, _ROWS, _DIM), jnp.float32),
        pltpu.VMEM((_BPW, _DIM), jnp.float32),
        pltpu.SemaphoreType.DMA,
        pltpu.SemaphoreType.DMA,
    ],
)
def _sc_sum(x_hbm, table_hbm, out_hbm, idx_v, rows_v, sums_v, sem0, sem1):
    wid = lax.axis_index("s") * _NC + lax.axis_index("c")
    base = wid * _BPW
    xbase = base * _L
    sems = (sem0, sem1)

    def start(k, buf):
        pltpu.sync_copy(x_hbm.at[pl.ds(xbase + k * _ROWS, _ROWS)],
                        idx_v.at[buf])

        def remap(i, carry):
            v = idx_v[buf, pl.ds(i * _LANES, _LANES)]
            idx_v[buf, pl.ds(i * _LANES, _LANES)] = jnp.where(
                v < _HALFV, 2 * v, 2 * v - (2 * _HALFV - 1))
            return carry

        lax.fori_loop(0, _ROWS // _LANES, remap, 0)
        pltpu.async_copy(table_hbm.at[idx_v.at[buf]], rows_v.at[buf],
                         sems[buf])

    def wait(buf):
        pltpu.make_async_copy(table_hbm.at[idx_v.at[buf]], rows_v.at[buf],
                              sems[buf]).wait()

    def sum_chunk(k, buf):
        def srow(j, accs):
            out = []
            for c in range(_CH):
                for g in range(_NG):
                    a = accs[c * _NG + g]
                    a = a + rows_v[buf, c * _L + j, pl.ds(g * _LANES, _LANES)]
                    out.append(a)
            return tuple(out)

        accs = lax.fori_loop(
            0, _L, srow,
            tuple(jnp.zeros((_LANES,), jnp.float32)
                  for _ in range(_CH * _NG)))
        for c in range(_CH):
            for g in range(_NG):
                sums_v[k * _CH + c, pl.ds(g * _LANES, _LANES)] = \
                    accs[c * _NG + g]

    start(0, 0)

    def body(t, carry):
        start(2 * t + 1, 1)
        wait(0)
        sum_chunk(2 * t, 0)

        @pl.when(t < _NCHUNK // 2 - 1)
        def _():
            start(2 * t + 2, 0)

        wait(1)
        sum_chunk(2 * t + 1, 1)
        return carry

    lax.fori_loop(0, _NCHUNK // 2, body, 0)
    pltpu.sync_copy(sums_v, out_hbm.at[pl.ds(base, _BPW)])


def _mlp_body(s_ref, x_ref, w1_ref, b1_ref, w2_ref, b2_ref, w3_ref, b3_ref,
              o_ref):
    hp = lax.Precision.HIGHEST
    mask = (x_ref[...] != 0).astype(jnp.float32)
    lengths = jnp.maximum(jnp.sum(mask, axis=1, keepdims=True), 1.0)
    pooled = s_ref[...] / lengths
    h = jnp.dot(pooled, w1_ref[...], precision=hp,
                preferred_element_type=jnp.float32) + b1_ref[...]
    h = jnp.maximum(h, 0.0)
    h = jnp.dot(h, w2_ref[...], precision=hp,
                preferred_element_type=jnp.float32) + b2_ref[...]
    h = jnp.maximum(h, 0.0)
    o_ref[...] = jnp.dot(h, w3_ref[...], precision=hp,
                         preferred_element_type=jnp.float32) + b3_ref[...]


def kernel(x, table, W1, b1, W2, b2, W3, b3):
    tableT = table.T
    conv = pl.pallas_call(
        _conv_body,
        grid=(_NBLK,),
        in_specs=[
            pl.BlockSpec((_DIM, _BQ), lambda i: (0, i)),
            pl.BlockSpec((_DIM, _BQ),
                         lambda i: (0, jnp.minimum(i + _NBLK, _LASTB))),
        ],
        out_specs=pl.BlockSpec((_BQ, 2 * _DIM), lambda i: (i, 0)),
        out_shape=jax.ShapeDtypeStruct((_HALFV, 2 * _DIM), jnp.float32),
    )(tableT, tableT)
    packed = conv.reshape(2 * _HALFV, _DIM)

    sums = _sc_sum(x.reshape(-1), packed)

    nblk = 4
    bm = _B // nblk
    out = pl.pallas_call(
        _mlp_body,
        grid=(nblk,),
        in_specs=[
            pl.BlockSpec((bm, _DIM), lambda i: (i, 0)),
            pl.BlockSpec((bm, _L), lambda i: (i, 0)),
            pl.BlockSpec((_DIM, _H1), lambda i: (0, 0)),
            pl.BlockSpec((1, _H1), lambda i: (0, 0)),
            pl.BlockSpec((_H1, _H2), lambda i: (0, 0)),
            pl.BlockSpec((1, _H2), lambda i: (0, 0)),
            pl.BlockSpec((_H2, _OUTP), lambda i: (0, 0)),
            pl.BlockSpec((1, _OUTP), lambda i: (0, 0)),
        ],
        out_specs=pl.BlockSpec((bm, _OUTP), lambda i: (i, 0)),
        out_shape=jax.ShapeDtypeStruct((_B, _OUTP), jnp.float32),
    )(sums, x, W1, b1.reshape(1, _H1), W2, b2.reshape(1, _H2),
      jnp.pad(W3, ((0, 0), (0, _OUTP - W3.shape[1]))),
      jnp.pad(b3, (0, _OUTP - b3.shape[0])).reshape(1, _OUTP))
    return out[:, :2]


# speedup vs baseline: 4.0003x; 1.0002x over previous
import functools

import jax
import jax.numpy as jnp
from jax import lax
from jax.experimental import pallas as pl
from jax.experimental.pallas import tpu as pltpu
from jax.experimental.pallas import tpu_sc as plsc

_DIM = 64
_B = 4096
_L = 200
_V = 1000000
_NC, _NS, _LANES = 2, 16, 16
_NW = _NC * _NS
_BPW = _B // _NW
_H1, _H2, _OUTP = 256, 128, 128
_NG = _DIM // _LANES
_CH = 4
_ROWS = _CH * _L
_NCHUNK = _BPW // _CH

_HALFV = 512000
_BQ = 4096
_NBLK = _HALFV // _BQ
_LASTB = (_V + _BQ - 1) // _BQ - 1


def _conv_body(a_ref, b_ref, o_ref):
    two = 2 * _DIM
    eye = (lax.broadcasted_iota(jnp.int32, (two, two), 0) ==
           lax.broadcasted_iota(jnp.int32, (two, two), 1)
           ).astype(jnp.float32)
    dn = (((0,), (0,)), ((), ()))
    hp = lax.Precision.DEFAULT
    s = jnp.concatenate([a_ref[...], b_ref[...]], axis=0)
    o_ref[...] = lax.dot_general(s, eye, dn, precision=hp,
                                 preferred_element_type=jnp.float32)


@functools.partial(
    pl.kernel,
    mesh=plsc.VectorSubcoreMesh(core_axis_name="c", subcore_axis_name="s"),
    compiler_params=pltpu.CompilerParams(use_tc_tiling_on_sc=False,
                                         needs_layout_passes=False),
    out_type=jax.ShapeDtypeStruct((_B, _DIM), jnp.float32),
    scratch_types=[
        pltpu.VMEM((2, _ROWS), jnp.int32),
        pltpu.VMEM((2, _ROWS, _DIM), jnp.float32),
        pltpu.VMEM((_BPW, _DIM), jnp.float32),
        pltpu.SemaphoreType.DMA,
        pltpu.SemaphoreType.DMA,
    ],
)
def _sc_sum(x_hbm, table_hbm, out_hbm, idx_v, rows_v, sums_v, sem0, sem1):
    wid = lax.axis_index("s") * _NC + lax.axis_index("c")
    base = wid * _BPW
    xbase = base * _L
    sems = (sem0, sem1)

    def start(k, buf):
        pltpu.sync_copy(x_hbm.at[pl.ds(xbase + k * _ROWS, _ROWS)],
                        idx_v.at[buf])

        def remap(i, carry):
            v = idx_v[buf, pl.ds(i * _LANES, _LANES)]
            idx_v[buf, pl.ds(i * _LANES, _LANES)] = jnp.where(
                v < _HALFV, 2 * v, 2 * v - (2 * _HALFV - 1))
            return carry

        lax.fori_loop(0, _ROWS // _LANES, remap, 0)
        pltpu.async_copy(table_hbm.at[idx_v.at[buf]], rows_v.at[buf],
                         sems[buf])

    def wait(buf):
        pltpu.make_async_copy(table_hbm.at[idx_v.at[buf]], rows_v.at[buf],
                              sems[buf]).wait()

    def sum_chunk(k, buf):
        def srow(j, accs):
            out = []
            for c in range(_CH):
                for g in range(_NG):
                    a = accs[c * _NG + g]
                    a = a + rows_v[buf, c * _L + j, pl.ds(g * _LANES, _LANES)]
                    out.append(a)
            return tuple(out)

        accs = lax.fori_loop(
            0, _L, srow,
            tuple(jnp.zeros((_LANES,), jnp.float32)
                  for _ in range(_CH * _NG)))
        for c in range(_CH):
            for g in range(_NG):
                sums_v[k * _CH + c, pl.ds(g * _LANES, _LANES)] = \
                    accs[c * _NG + g]

    start(0, 0)

    def body(t, carry):
        start(2 * t + 1, 1)
        wait(0)
        sum_chunk(2 * t, 0)

        @pl.when(t < _NCHUNK // 2 - 1)
        def _():
            start(2 * t + 2, 0)

        wait(1)
        sum_chunk(2 * t + 1, 1)
        return carry

    lax.fori_loop(0, _NCHUNK // 2, body, 0)
    pltpu.sync_copy(sums_v, out_hbm.at[pl.ds(base, _BPW)])


def _mlp_body(s_ref, x_ref, w1_ref, b1_ref, w2_ref, b2_ref, w3_ref, b3_ref,
              o_ref):
    hp = lax.Precision.HIGHEST
    mask = (x_ref[...] != 0).astype(jnp.float32)
    lengths = jnp.maximum(jnp.sum(mask, axis=1, keepdims=True), 1.0)
    pooled = s_ref[...] / lengths
    h = jnp.dot(pooled, w1_ref[...], precision=hp,
                preferred_element_type=jnp.float32) + b1_ref[...]
    h = jnp.maximum(h, 0.0)
    h = jnp.dot(h, w2_ref[...], precision=hp,
                preferred_element_type=jnp.float32) + b2_ref[...]
    h = jnp.maximum(h, 0.0)
    o_ref[...] = jnp.dot(h, w3_ref[...], precision=hp,
                         preferred_element_type=jnp.float32) + b3_ref[...]


def kernel(x, table, W1, b1, W2, b2, W3, b3):
    tableT = table.T
    conv = pl.pallas_call(
        _conv_body,
        grid=(_NBLK,),
        in_specs=[
            pl.BlockSpec((_DIM, _BQ), lambda i: (0, i)),
            pl.BlockSpec((_DIM, _BQ),
                         lambda i: (0, jnp.minimum(i + _NBLK, _LASTB))),
        ],
        out_specs=pl.BlockSpec((_BQ, 2 * _DIM), lambda i: (i, 0)),
        out_shape=jax.ShapeDtypeStruct((_HALFV, 2 * _DIM), jnp.float32),
    )(tableT, tableT)
    packed = conv.reshape(2 * _HALFV, _DIM)

    sums = _sc_sum(x.reshape(-1), packed)

    nblk = 4
    bm = _B // nblk
    out = pl.pallas_call(
        _mlp_body,
        grid=(nblk,),
        in_specs=[
            pl.BlockSpec((bm, _DIM), lambda i: (i, 0)),
            pl.BlockSpec((bm, _L), lambda i: (i, 0)),
            pl.BlockSpec((_DIM, _H1), lambda i: (0, 0)),
            pl.BlockSpec((1, _H1), lambda i: (0, 0)),
            pl.BlockSpec((_H1, _H2), lambda i: (0, 0)),
            pl.BlockSpec((1, _H2), lambda i: (0, 0)),
            pl.BlockSpec((_H2, _OUTP), lambda i: (0, 0)),
            pl.BlockSpec((1, _OUTP), lambda i: (0, 0)),
        ],
        out_specs=pl.BlockSpec((bm, _OUTP), lambda i: (i, 0)),
        out_shape=jax.ShapeDtypeStruct((_B, _OUTP), jnp.float32),
    )(sums, x, W1, b1.reshape(1, _H1), W2, b2.reshape(1, _H2),
      jnp.pad(W3, ((0, 0), (0, _OUTP - W3.shape[1]))),
      jnp.pad(b3, (0, _OUTP - b3.shape[0])).reshape(1, _OUTP))
    return out[:, :2]


# speedup vs baseline: 4.4016x; 1.1003x over previous
import functools

import jax
import jax.numpy as jnp
from jax import lax
from jax.experimental import pallas as pl
from jax.experimental.pallas import tpu as pltpu
from jax.experimental.pallas import tpu_sc as plsc

_DIM = 64
_B = 4096
_L = 200
_V = 1000000
_NC, _NS, _LANES = 2, 16, 16
_NW = _NC * _NS
_BPW = _B // _NW
_H1, _H2, _OUTP = 256, 128, 128
_NG = _DIM // _LANES
_CH = 4
_ROWS = _CH * _L
_NCHUNK = _BPW // _CH

_HALFV = 512000
_BQ = 10240
_NBLK = _HALFV // _BQ
_LASTB = (_V + _BQ - 1) // _BQ - 1


def _conv_body(a_ref, b_ref, o_ref):
    two = 2 * _DIM
    eye = (lax.broadcasted_iota(jnp.int32, (two, two), 0) ==
           lax.broadcasted_iota(jnp.int32, (two, two), 1)
           ).astype(jnp.float32)
    dn = (((0,), (0,)), ((), ()))
    hp = lax.Precision.DEFAULT
    s = jnp.concatenate([a_ref[...], b_ref[...]], axis=0)
    o_ref[...] = lax.dot_general(s, eye, dn, precision=hp,
                                 preferred_element_type=jnp.float32)


@functools.partial(
    pl.kernel,
    mesh=plsc.VectorSubcoreMesh(core_axis_name="c", subcore_axis_name="s"),
    compiler_params=pltpu.CompilerParams(use_tc_tiling_on_sc=False,
                                         needs_layout_passes=False),
    out_type=jax.ShapeDtypeStruct((_B, _DIM), jnp.float32),
    scratch_types=[
        pltpu.VMEM((2, _ROWS), jnp.int32),
        pltpu.VMEM((2, _ROWS, _DIM), jnp.float32),
        pltpu.VMEM((_BPW, _DIM), jnp.float32),
        pltpu.SemaphoreType.DMA,
        pltpu.SemaphoreType.DMA,
    ],
)
def _sc_sum(x_hbm, table_hbm, out_hbm, idx_v, rows_v, sums_v, sem0, sem1):
    wid = lax.axis_index("s") * _NC + lax.axis_index("c")
    base = wid * _BPW
    xbase = base * _L
    sems = (sem0, sem1)

    def start(k, buf):
        pltpu.sync_copy(x_hbm.at[pl.ds(xbase + k * _ROWS, _ROWS)],
                        idx_v.at[buf])

        def remap(i, carry):
            v = idx_v[buf, pl.ds(i * _LANES, _LANES)]
            idx_v[buf, pl.ds(i * _LANES, _LANES)] = jnp.where(
                v < _HALFV, 2 * v, 2 * v - (2 * _HALFV - 1))
            return carry

        lax.fori_loop(0, _ROWS // _LANES, remap, 0)
        pltpu.async_copy(table_hbm.at[idx_v.at[buf]], rows_v.at[buf],
                         sems[buf])

    def wait(buf):
        pltpu.make_async_copy(table_hbm.at[idx_v.at[buf]], rows_v.at[buf],
                              sems[buf]).wait()

    def sum_chunk(k, buf):
        def srow(j, accs):
            out = []
            for c in range(_CH):
                for g in range(_NG):
                    a = accs[c * _NG + g]
                    a = a + rows_v[buf, c * _L + j, pl.ds(g * _LANES, _LANES)]
                    out.append(a)
            return tuple(out)

        accs = lax.fori_loop(
            0, _L, srow,
            tuple(jnp.zeros((_LANES,), jnp.float32)
                  for _ in range(_CH * _NG)))
        for c in range(_CH):
            for g in range(_NG):
                sums_v[k * _CH + c, pl.ds(g * _LANES, _LANES)] = \
                    accs[c * _NG + g]

    start(0, 0)

    def body(t, carry):
        start(2 * t + 1, 1)
        wait(0)
        sum_chunk(2 * t, 0)

        @pl.when(t < _NCHUNK // 2 - 1)
        def _():
            start(2 * t + 2, 0)

        wait(1)
        sum_chunk(2 * t + 1, 1)
        return carry

    lax.fori_loop(0, _NCHUNK // 2, body, 0)
    pltpu.sync_copy(sums_v, out_hbm.at[pl.ds(base, _BPW)])


def _mlp_body(s_ref, x_ref, w1_ref, b1_ref, w2_ref, b2_ref, w3_ref, b3_ref,
              o_ref):
    hp = lax.Precision.HIGHEST
    mask = (x_ref[...] != 0).astype(jnp.float32)
    lengths = jnp.maximum(jnp.sum(mask, axis=1, keepdims=True), 1.0)
    pooled = s_ref[...] / lengths
    h = jnp.dot(pooled, w1_ref[...], precision=hp,
                preferred_element_type=jnp.float32) + b1_ref[...]
    h = jnp.maximum(h, 0.0)
    h = jnp.dot(h, w2_ref[...], precision=hp,
                preferred_element_type=jnp.float32) + b2_ref[...]
    h = jnp.maximum(h, 0.0)
    o_ref[...] = jnp.dot(h, w3_ref[...], precision=hp,
                         preferred_element_type=jnp.float32) + b3_ref[...]


def kernel(x, table, W1, b1, W2, b2, W3, b3):
    tableT = table.T
    conv = pl.pallas_call(
        _conv_body,
        grid=(_NBLK,),
        in_specs=[
            pl.BlockSpec((_DIM, _BQ), lambda i: (0, i)),
            pl.BlockSpec((_DIM, _BQ),
                         lambda i: (0, jnp.minimum(i + _NBLK, _LASTB))),
        ],
        out_specs=pl.BlockSpec((_BQ, 2 * _DIM), lambda i: (i, 0)),
        out_shape=jax.ShapeDtypeStruct((_HALFV, 2 * _DIM), jnp.float32),
    )(tableT, tableT)
    packed = conv.reshape(2 * _HALFV, _DIM)

    sums = _sc_sum(x.reshape(-1), packed)

    nblk = 4
    bm = _B // nblk
    out = pl.pallas_call(
        _mlp_body,
        grid=(nblk,),
        in_specs=[
            pl.BlockSpec((bm, _DIM), lambda i: (i, 0)),
            pl.BlockSpec((bm, _L), lambda i: (i, 0)),
            pl.BlockSpec((_DIM, _H1), lambda i: (0, 0)),
            pl.BlockSpec((1, _H1), lambda i: (0, 0)),
            pl.BlockSpec((_H1, _H2), lambda i: (0, 0)),
            pl.BlockSpec((1, _H2), lambda i: (0, 0)),
            pl.BlockSpec((_H2, _OUTP), lambda i: (0, 0)),
            pl.BlockSpec((1, _OUTP), lambda i: (0, 0)),
        ],
        out_specs=pl.BlockSpec((bm, _OUTP), lambda i: (i, 0)),
        out_shape=jax.ShapeDtypeStruct((_B, _OUTP), jnp.float32),
    )(sums, x, W1, b1.reshape(1, _H1), W2, b2.reshape(1, _H2),
      jnp.pad(W3, ((0, 0), (0, _OUTP - W3.shape[1]))),
      jnp.pad(b3, (0, _OUTP - b3.shape[0])).reshape(1, _OUTP))
    return out[:, :2]


# speedup vs baseline: 4.4411x; 1.0090x over previous
import functools

import jax
import jax.numpy as jnp
from jax import lax
from jax.experimental import pallas as pl
from jax.experimental.pallas import tpu as pltpu
from jax.experimental.pallas import tpu_sc as plsc

_DIM = 64
_B = 4096
_L = 200
_V = 1000000
_NC, _NS, _LANES = 2, 16, 16
_NW = _NC * _NS
_BPW = _B // _NW
_H1, _H2, _OUTP = 256, 128, 128
_NG = _DIM // _LANES
_CH = 4
_ROWS = _CH * _L
_NCHUNK = _BPW // _CH

_HALFV = 512000
_BQ = 20480
_NBLK = _HALFV // _BQ
_LASTB = (_V + _BQ - 1) // _BQ - 1


def _conv_body(a_ref, b_ref, o_ref):
    two = 2 * _DIM
    eye = (lax.broadcasted_iota(jnp.int32, (two, two), 0) ==
           lax.broadcasted_iota(jnp.int32, (two, two), 1)
           ).astype(jnp.float32)
    dn = (((0,), (0,)), ((), ()))
    hp = lax.Precision.DEFAULT
    s = jnp.concatenate([a_ref[...], b_ref[...]], axis=0)
    o_ref[...] = lax.dot_general(s, eye, dn, precision=hp,
                                 preferred_element_type=jnp.float32)


@functools.partial(
    pl.kernel,
    mesh=plsc.VectorSubcoreMesh(core_axis_name="c", subcore_axis_name="s"),
    compiler_params=pltpu.CompilerParams(use_tc_tiling_on_sc=False,
                                         needs_layout_passes=False),
    out_type=jax.ShapeDtypeStruct((_B, _DIM), jnp.float32),
    scratch_types=[
        pltpu.VMEM((2, _ROWS), jnp.int32),
        pltpu.VMEM((2, _ROWS, _DIM), jnp.float32),
        pltpu.VMEM((_BPW, _DIM), jnp.float32),
        pltpu.SemaphoreType.DMA,
        pltpu.SemaphoreType.DMA,
    ],
)
def _sc_sum(x_hbm, table_hbm, out_hbm, idx_v, rows_v, sums_v, sem0, sem1):
    wid = lax.axis_index("s") * _NC + lax.axis_index("c")
    base = wid * _BPW
    xbase = base * _L
    sems = (sem0, sem1)

    def start(k, buf):
        pltpu.sync_copy(x_hbm.at[pl.ds(xbase + k * _ROWS, _ROWS)],
                        idx_v.at[buf])

        def remap(i, carry):
            v = idx_v[buf, pl.ds(i * _LANES, _LANES)]
            idx_v[buf, pl.ds(i * _LANES, _LANES)] = jnp.where(
                v < _HALFV, 2 * v, 2 * v - (2 * _HALFV - 1))
            return carry

        lax.fori_loop(0, _ROWS // _LANES, remap, 0)
        pltpu.async_copy(table_hbm.at[idx_v.at[buf]], rows_v.at[buf],
                         sems[buf])

    def wait(buf):
        pltpu.make_async_copy(table_hbm.at[idx_v.at[buf]], rows_v.at[buf],
                              sems[buf]).wait()

    def sum_chunk(k, buf):
        def srow(j, accs):
            out = []
            for c in range(_CH):
                for g in range(_NG):
                    a = accs[c * _NG + g]
                    a = a + rows_v[buf, c * _L + j, pl.ds(g * _LANES, _LANES)]
                    out.append(a)
            return tuple(out)

        accs = lax.fori_loop(
            0, _L, srow,
            tuple(jnp.zeros((_LANES,), jnp.float32)
                  for _ in range(_CH * _NG)))
        for c in range(_CH):
            for g in range(_NG):
                sums_v[k * _CH + c, pl.ds(g * _LANES, _LANES)] = \
                    accs[c * _NG + g]

    start(0, 0)

    def body(t, carry):
        start(2 * t + 1, 1)
        wait(0)
        sum_chunk(2 * t, 0)

        @pl.when(t < _NCHUNK // 2 - 1)
        def _():
            start(2 * t + 2, 0)

        wait(1)
        sum_chunk(2 * t + 1, 1)
        return carry

    lax.fori_loop(0, _NCHUNK // 2, body, 0)
    pltpu.sync_copy(sums_v, out_hbm.at[pl.ds(base, _BPW)])


def _mlp_body(s_ref, x_ref, w1_ref, b1_ref, w2_ref, b2_ref, w3_ref, b3_ref,
              o_ref):
    hp = lax.Precision.HIGHEST
    mask = (x_ref[...] != 0).astype(jnp.float32)
    lengths = jnp.maximum(jnp.sum(mask, axis=1, keepdims=True), 1.0)
    pooled = s_ref[...] / lengths
    h = jnp.dot(pooled, w1_ref[...], precision=hp,
                preferred_element_type=jnp.float32) + b1_ref[...]
    h = jnp.maximum(h, 0.0)
    h = jnp.dot(h, w2_ref[...], precision=hp,
                preferred_element_type=jnp.float32) + b2_ref[...]
    h = jnp.maximum(h, 0.0)
    o_ref[...] = jnp.dot(h, w3_ref[...], precision=hp,
                         preferred_element_type=jnp.float32) + b3_ref[...]


def kernel(x, table, W1, b1, W2, b2, W3, b3):
    tableT = table.T
    conv = pl.pallas_call(
        _conv_body,
        grid=(_NBLK,),
        in_specs=[
            pl.BlockSpec((_DIM, _BQ), lambda i: (0, i)),
            pl.BlockSpec((_DIM, _BQ),
                         lambda i: (0, jnp.minimum(i + _NBLK, _LASTB))),
        ],
        out_specs=pl.BlockSpec((_BQ, 2 * _DIM), lambda i: (i, 0)),
        out_shape=jax.ShapeDtypeStruct((_HALFV, 2 * _DIM), jnp.float32),
    )(tableT, tableT)
    packed = conv.reshape(2 * _HALFV, _DIM)

    sums = _sc_sum(x.reshape(-1), packed)

    nblk = 4
    bm = _B // nblk
    out = pl.pallas_call(
        _mlp_body,
        grid=(nblk,),
        in_specs=[
            pl.BlockSpec((bm, _DIM), lambda i: (i, 0)),
            pl.BlockSpec((bm, _L), lambda i: (i, 0)),
            pl.BlockSpec((_DIM, _H1), lambda i: (0, 0)),
            pl.BlockSpec((1, _H1), lambda i: (0, 0)),
            pl.BlockSpec((_H1, _H2), lambda i: (0, 0)),
            pl.BlockSpec((1, _H2), lambda i: (0, 0)),
            pl.BlockSpec((_H2, _OUTP), lambda i: (0, 0)),
            pl.BlockSpec((1, _OUTP), lambda i: (0, 0)),
        ],
        out_specs=pl.BlockSpec((bm, _OUTP), lambda i: (i, 0)),
        out_shape=jax.ShapeDtypeStruct((_B, _OUTP), jnp.float32),
    )(sums, x, W1, b1.reshape(1, _H1), W2, b2.reshape(1, _H2),
      jnp.pad(W3, ((0, 0), (0, _OUTP - W3.shape[1]))),
      jnp.pad(b3, (0, _OUTP - b3.shape[0])).reshape(1, _OUTP))
    return out[:, :2]


# speedup vs baseline: 4.6217x; 1.0407x over previous
import functools

import jax
import jax.numpy as jnp
from jax import lax
from jax.experimental import pallas as pl
from jax.experimental.pallas import tpu as pltpu
from jax.experimental.pallas import tpu_sc as plsc

_DIM = 64
_B = 4096
_L = 200
_V = 1000000
_NC, _NS, _LANES = 2, 16, 16
_NW = _NC * _NS
_BPW = _B // _NW
_H1, _H2, _OUTP = 256, 128, 128
_NG = _DIM // _LANES
_CH = 4
_ROWS = _CH * _L
_NCHUNK = _BPW // _CH

_HALFV = 512000
_BQ = 20480
_NBLK = _HALFV // _BQ
_LASTB = (_V + _BQ - 1) // _BQ - 1


def _conv_body(a_ref, b_ref, o_ref):
    two = 2 * _DIM
    eye = (lax.broadcasted_iota(jnp.int32, (two, two), 0) ==
           lax.broadcasted_iota(jnp.int32, (two, two), 1)
           ).astype(jnp.float32)
    dn = (((0,), (0,)), ((), ()))
    hp = lax.Precision.DEFAULT
    s = jnp.concatenate([a_ref[...], b_ref[...]], axis=0)
    o_ref[...] = lax.dot_general(s, eye, dn, precision=hp,
                                 preferred_element_type=jnp.float32)


@functools.partial(
    pl.kernel,
    mesh=plsc.VectorSubcoreMesh(core_axis_name="c", subcore_axis_name="s"),
    compiler_params=pltpu.CompilerParams(use_tc_tiling_on_sc=False,
                                         needs_layout_passes=False),
    out_type=jax.ShapeDtypeStruct((_B, _DIM), jnp.float32),
    scratch_types=[
        pltpu.VMEM((4, _ROWS), jnp.int32),
        pltpu.VMEM((2, _ROWS, _DIM), jnp.float32),
        pltpu.VMEM((_BPW, _DIM), jnp.float32),
        pltpu.SemaphoreType.DMA,
        pltpu.SemaphoreType.DMA,
        pltpu.SemaphoreType.DMA,
        pltpu.SemaphoreType.DMA,
        pltpu.SemaphoreType.DMA,
        pltpu.SemaphoreType.DMA,
    ],
)
def _sc_pool(x_hbm, table_hbm, out_hbm, idx_v, rows_v, pooled_v,
             si0, si1, si2, si3, sr0, sr1):
    wid = lax.axis_index("s") * _NC + lax.axis_index("c")
    base = wid * _BPW
    xbase = base * _L
    isems = (si0, si1, si2, si3)
    rsems = (sr0, sr1)

    def idx_start(k, ib):
        pltpu.async_copy(x_hbm.at[pl.ds(xbase + k * _ROWS, _ROWS)],
                         idx_v.at[ib], isems[ib])

    def idx_wait(k, ib):
        pltpu.make_async_copy(x_hbm.at[pl.ds(xbase + k * _ROWS, _ROWS)],
                              idx_v.at[ib], isems[ib]).wait()

    def gather_start(k, ib, rb):
        def remap(i, carry):
            v = idx_v[ib, pl.ds(i * _LANES, _LANES)]
            idx_v[ib, pl.ds(i * _LANES, _LANES)] = jnp.where(
                v < _HALFV, 2 * v, 2 * v - (2 * _HALFV - 1))
            return carry

        lax.fori_loop(0, _ROWS // _LANES, remap, 0)
        pltpu.async_copy(table_hbm.at[idx_v.at[ib]], rows_v.at[rb],
                         rsems[rb])

    def rows_wait(rb):
        pltpu.make_async_copy(table_hbm.at[idx_v.at[0]], rows_v.at[rb],
                              rsems[rb]).wait()

    def pooled_chunk(k, ib, rb):
        def srow(j, accs):
            out = []
            for c in range(_CH):
                for g in range(_NG):
                    a = accs[c * _NG + g]
                    a = a + rows_v[rb, c * _L + j, pl.ds(g * _LANES, _LANES)]
                    out.append(a)
            return tuple(out)

        accs = lax.fori_loop(
            0, _L, srow,
            tuple(jnp.zeros((_LANES,), jnp.float32)
                  for _ in range(_CH * _NG)))

        low8 = lax.broadcasted_iota(jnp.int32, (_LANES,), 0) < 8
        cnts = []
        for c in range(_CH):
            lo_e = c * _L
            hi_e = lo_e + _L
            v0 = lo_e // _LANES
            v1 = hi_e // _LANES
            cnt = jnp.zeros((_LANES,), jnp.int32)
            if lo_e % _LANES:
                m = (idx_v[ib, pl.ds(v0 * _LANES, _LANES)] != 0) & (~low8)
                cnt = cnt + plsc.all_reduce_population_count(m)
                v0 += 1
            for i in range(v0, v1):
                m = idx_v[ib, pl.ds(i * _LANES, _LANES)] != 0
                cnt = cnt + plsc.all_reduce_population_count(m)
            if hi_e % _LANES:
                m = (idx_v[ib, pl.ds(v1 * _LANES, _LANES)] != 0) & low8
                cnt = cnt + plsc.all_reduce_population_count(m)
            cnts.append(cnt)

        for c in range(_CH):
            inv = 1.0 / jnp.maximum(cnts[c].astype(jnp.float32), 1.0)
            for g in range(_NG):
                pooled_v[k * _CH + c, pl.ds(g * _LANES, _LANES)] = \
                    accs[c * _NG + g] * inv

    idx_start(0, 0)
    idx_start(1, 1)
    idx_start(2, 2)
    idx_wait(0, 0)
    gather_start(0, 0, 0)

    def body(t, carry):
        c0 = 4 * t
        idx_wait(c0 + 1, 1)
        gather_start(c0 + 1, 1, 1)
        idx_start(c0 + 3, 3)
        rows_wait(0)
        pooled_chunk(c0, 0, 0)
        idx_wait(c0 + 2, 2)
        gather_start(c0 + 2, 2, 0)

        @pl.when(c0 + 4 < _NCHUNK)
        def _():
            idx_start(c0 + 4, 0)

        rows_wait(1)
        pooled_chunk(c0 + 1, 1, 1)
        idx_wait(c0 + 3, 3)
        gather_start(c0 + 3, 3, 1)

        @pl.when(c0 + 5 < _NCHUNK)
        def _():
            idx_start(c0 + 5, 1)

        rows_wait(0)
        pooled_chunk(c0 + 2, 2, 0)

        @pl.when(c0 + 6 < _NCHUNK)
        def _():
            idx_start(c0 + 6, 2)

        @pl.when(c0 + 4 < _NCHUNK)
        def _():
            idx_wait(c0 + 4, 0)
            gather_start(c0 + 4, 0, 0)

        rows_wait(1)
        pooled_chunk(c0 + 3, 3, 1)
        return carry

    lax.fori_loop(0, _NCHUNK // 4, body, 0)
    pltpu.sync_copy(pooled_v, out_hbm.at[pl.ds(base, _BPW)])


def _mlp_body(s_ref, w1_ref, b1_ref, w2_ref, b2_ref, w3_ref, b3_ref,
              o_ref):
    hp = lax.Precision.HIGHEST
    pooled = s_ref[...]
    h = jnp.dot(pooled, w1_ref[...], precision=hp,
                preferred_element_type=jnp.float32) + b1_ref[...]
    h = jnp.maximum(h, 0.0)
    h = jnp.dot(h, w2_ref[...], precision=hp,
                preferred_element_type=jnp.float32) + b2_ref[...]
    h = jnp.maximum(h, 0.0)
    o_ref[...] = jnp.dot(h, w3_ref[...], precision=hp,
                         preferred_element_type=jnp.float32) + b3_ref[...]


def kernel(x, table, W1, b1, W2, b2, W3, b3):
    tableT = table.T
    conv = pl.pallas_call(
        _conv_body,
        grid=(_NBLK,),
        in_specs=[
            pl.BlockSpec((_DIM, _BQ), lambda i: (0, i)),
            pl.BlockSpec((_DIM, _BQ),
                         lambda i: (0, jnp.minimum(i + _NBLK, _LASTB))),
        ],
        out_specs=pl.BlockSpec((_BQ, 2 * _DIM), lambda i: (i, 0)),
        out_shape=jax.ShapeDtypeStruct((_HALFV, 2 * _DIM), jnp.float32),
    )(tableT, tableT)
    packed = conv.reshape(2 * _HALFV, _DIM)

    pooled = _sc_pool(x.reshape(-1), packed)

    nblk = 4
    bm = _B // nblk
    out = pl.pallas_call(
        _mlp_body,
        grid=(nblk,),
        in_specs=[
            pl.BlockSpec((bm, _DIM), lambda i: (i, 0)),
            pl.BlockSpec((_DIM, _H1), lambda i: (0, 0)),
            pl.BlockSpec((1, _H1), lambda i: (0, 0)),
            pl.BlockSpec((_H1, _H2), lambda i: (0, 0)),
            pl.BlockSpec((1, _H2), lambda i: (0, 0)),
            pl.BlockSpec((_H2, _OUTP), lambda i: (0, 0)),
            pl.BlockSpec((1, _OUTP), lambda i: (0, 0)),
        ],
        out_specs=pl.BlockSpec((bm, _OUTP), lambda i: (i, 0)),
        out_shape=jax.ShapeDtypeStruct((_B, _OUTP), jnp.float32),
    )(pooled, W1, b1.reshape(1, _H1), W2, b2.reshape(1, _H2),
      jnp.pad(W3, ((0, 0), (0, _OUTP - W3.shape[1]))),
      jnp.pad(b3, (0, _OUTP - b3.shape[0])).reshape(1, _OUTP))
    return out[:, :2]
